# Initial kernel scaffold; baseline (speedup 1.0000x reference)
import jax
import jax.numpy as jnp
from jax.experimental import pallas as pl


def kernel(scores, bbox_deltas, im_info, anchors, ids):
    raise NotImplementedError("write your pallas kernel here")


# speedup vs baseline: 52.7402x; 52.7402x over previous
import functools

import jax
import jax.numpy as jnp
from jax.experimental import pallas as pl
from jax.experimental.pallas import tpu as pltpu

_POST_NMS_TOP_N = 1000
_NMS_THRESH = 0.7
_BS = 128
_NEG = -3e38


def _nms_kernel(sc_ref, dl_ref, an_ref, id_ref, lim_ref, out_ref, s_ref, k_ref,
                *, n, np_, top_n, thresh):
    f32 = jnp.float32
    nb = np_ // _BS

    ax1 = an_ref[0:1, :]
    ay1 = an_ref[1:2, :]
    ax2 = an_ref[2:3, :]
    ay2 = an_ref[3:4, :]
    widths = ax2 - ax1 + 1.0
    heights = ay2 - ay1 + 1.0
    ctr_x = ax1 + 0.5 * widths
    ctr_y = ay1 + 0.5 * heights
    dx = dl_ref[0:1, :]
    dy = dl_ref[1:2, :]
    dw = dl_ref[2:3, :]
    dh = dl_ref[3:4, :]
    pred_ctr_x = dx * widths + ctr_x
    pred_ctr_y = dy * heights + ctr_y
    pred_w = jnp.exp(dw) * widths
    pred_h = jnp.exp(dh) * heights
    hlim = lim_ref[0:1, :]
    wlim = lim_ref[1:2, :]
    x1 = jnp.clip(pred_ctr_x - 0.5 * pred_w, 0.0, wlim)
    y1 = jnp.clip(pred_ctr_y - 0.5 * pred_h, 0.0, hlim)
    x2 = jnp.clip(pred_ctr_x + 0.5 * pred_w, 0.0, wlim)
    y2 = jnp.clip(pred_ctr_y + 0.5 * pred_h, 0.0, hlim)

    idxr = jax.lax.broadcasted_iota(jnp.int32, (1, np_), 1).astype(f32)
    validr = idxr < float(n)

    neg = jnp.full((1, np_), _NEG, f32)
    mc = jnp.maximum(
        jnp.maximum(jnp.max(jnp.where(validr, x1, neg)),
                    jnp.max(jnp.where(validr, y1, neg))),
        jnp.maximum(jnp.max(jnp.where(validr, x2, neg)),
                    jnp.max(jnp.where(validr, y2, neg)))) + 1.0

    off = id_ref[0:1, :] * mc
    sx1 = x1 + off
    sy1 = y1 + off
    sx2 = x2 + off
    sy2 = y2 + off
    areas = (sx2 - sx1 + 1.0) * (sy2 - sy1 + 1.0)
    sc = sc_ref[0:1, :]

    zero5 = jnp.zeros((5, np_), f32)
    s_ref[...] = jnp.concatenate(
        [sx1, sy1, sx2, sy2, areas, sc, idxr, x1, y1, x2, y2, zero5], axis=0)
    k_ref[...] = jnp.ones((1, np_), f32)

    def dom_tile(ib, keepj_row):
        cols = jnp.transpose(s_ref[:, pl.ds(pl.multiple_of(ib * _BS, _BS), _BS)])
        xi = cols[:, 0:1]
        yi = cols[:, 1:2]
        x2i = cols[:, 2:3]
        y2i = cols[:, 3:4]
        ai = cols[:, 4:5]
        si = cols[:, 5:6]
        ii = cols[:, 6:7]
        xx1 = jnp.maximum(xi, sx1)
        yy1 = jnp.maximum(yi, sy1)
        xx2 = jnp.minimum(x2i, sx2)
        yy2 = jnp.minimum(y2i, sy2)
        inter = (jnp.maximum(xx2 - xx1 + 1.0, 0.0) *
                 jnp.maximum(yy2 - yy1 + 1.0, 0.0))
        iou = inter / (ai + areas - inter)
        hi = (sc > si) | ((sc == si) & (idxr < ii))
        return (iou > thresh) & hi & keepj_row, cols

    def sweep_cond(st):
        return st[1] > 0.0

    def sweep_body(st):
        it = st[0]
        kj = k_ref[0:1, :]
        keepj = kj > 0.0

        def blk(ib, _):
            d, _cols = dom_tile(ib, keepj)
            supp = jnp.max(jnp.where(d, 1.0, 0.0), axis=1, keepdims=True)
            k_ref[0:1, pl.ds(pl.multiple_of(ib * _BS, _BS), _BS)] = (
                jnp.transpose(1.0 - supp))
            return 0

        jax.lax.fori_loop(0, nb, blk, 0)
        knew = k_ref[0:1, :]
        changed = jnp.max(jnp.abs(knew - kj))
        return (it + 1, changed)

    it_f, _ = jax.lax.while_loop(sweep_cond, sweep_body, (jnp.int32(0), f32(1.0)))
    keep = k_ref[0:1, :] > 0.0

    trow = jax.lax.broadcasted_iota(jnp.int32, (1, top_n), 1).astype(f32)

    def blk2(ib, acc):
        base = pl.multiple_of(ib * _BS, _BS)
        cols = jnp.transpose(s_ref[:, pl.ds(base, _BS)])
        si = cols[:, 5:6]
        ii = cols[:, 6:7]
        hi = (sc > si) | ((sc == si) & (idxr < ii))
        pos = jnp.sum(jnp.where(hi & keep, 1.0, 0.0), axis=1, keepdims=True)
        onehot = (pos == trow).astype(f32)
        krow = k_ref[0:1, pl.ds(base, _BS)]
        lane = jax.lax.broadcasted_iota(jnp.int32, (1, _BS), 1) + ib * _BS
        vrow = (lane < n) & (krow > 0.0)
        bj = s_ref[7:11, pl.ds(base, _BS)] * jnp.where(vrow, 1.0, 0.0)
        return acc + jax.lax.dot_general(
            bj, onehot, (((1,), (0,)), ((), ())),
            precision=jax.lax.Precision.HIGHEST, preferred_element_type=f32)

    out_t = jax.lax.fori_loop(0, nb, blk2, jnp.zeros((4, top_n), f32))
    brow = jnp.full((1, top_n), pl.program_id(0).astype(f32))
    out_ref[...] = jnp.transpose(jnp.concatenate([brow, out_t], axis=0))


def kernel(scores, bbox_deltas, im_info, anchors, ids):
    f32 = jnp.float32
    b, n = scores.shape
    np_ = ((n + _BS - 1) // _BS) * _BS
    pad = np_ - n

    sc = jnp.pad(scores.astype(f32), ((0, 0), (0, pad)),
                 constant_values=_NEG)[:, None, :]
    dl = jnp.pad(bbox_deltas.astype(f32),
                 ((0, 0), (0, pad), (0, 0))).transpose(0, 2, 1)
    an = jnp.pad(anchors.astype(f32), ((0, pad), (0, 0))).T
    idr = jnp.pad(ids.astype(f32), (0, pad), constant_values=-1.0)[None, :]
    lim = jnp.stack([im_info[:, 0] - 1.0, im_info[:, 1] - 1.0], axis=1)
    lim = jnp.broadcast_to(lim.astype(f32)[:, :, None], (b, 2, np_))

    kern = functools.partial(_nms_kernel, n=n, np_=np_,
                             top_n=_POST_NMS_TOP_N, thresh=_NMS_THRESH)
    return pl.pallas_call(
        kern,
        grid=(b,),
        in_specs=[
            pl.BlockSpec((None, 1, np_), lambda i: (i, 0, 0)),
            pl.BlockSpec((None, 4, np_), lambda i: (i, 0, 0)),
            pl.BlockSpec((4, np_), lambda i: (0, 0)),
            pl.BlockSpec((1, np_), lambda i: (0, 0)),
            pl.BlockSpec((None, 2, np_), lambda i: (i, 0, 0)),
        ],
        out_specs=pl.BlockSpec((None, _POST_NMS_TOP_N, 5), lambda i: (i, 0, 0)),
        out_shape=jax.ShapeDtypeStruct((b, _POST_NMS_TOP_N, 5), f32),
        scratch_shapes=[
            pltpu.VMEM((16, np_), f32),
            pltpu.VMEM((1, np_), f32),
        ],
    )(sc, dl, an, idr, lim)


# speedup vs baseline: 59.8440x; 1.1347x over previous
import functools

import jax
import jax.numpy as jnp
from jax.experimental import pallas as pl
from jax.experimental.pallas import tpu as pltpu

_POST_NMS_TOP_N = 1000
_NMS_THRESH = 0.7
_BS = 128
_NEG = -3e38


def _nms_kernel(sc_ref, dl_ref, an_ref, id_ref, lim_ref, out_ref, s_ref, k_ref,
                *, n, np_, top_n, thresh):
    f32 = jnp.float32
    nb = np_ // _BS

    ax1 = an_ref[0:1, :]
    ay1 = an_ref[1:2, :]
    ax2 = an_ref[2:3, :]
    ay2 = an_ref[3:4, :]
    widths = ax2 - ax1 + 1.0
    heights = ay2 - ay1 + 1.0
    ctr_x = ax1 + 0.5 * widths
    ctr_y = ay1 + 0.5 * heights
    dx = dl_ref[0:1, :]
    dy = dl_ref[1:2, :]
    dw = dl_ref[2:3, :]
    dh = dl_ref[3:4, :]
    pred_ctr_x = dx * widths + ctr_x
    pred_ctr_y = dy * heights + ctr_y
    pred_w = jnp.exp(dw) * widths
    pred_h = jnp.exp(dh) * heights
    hlim = lim_ref[0:1, :]
    wlim = lim_ref[1:2, :]
    x1 = jnp.clip(pred_ctr_x - 0.5 * pred_w, 0.0, wlim)
    y1 = jnp.clip(pred_ctr_y - 0.5 * pred_h, 0.0, hlim)
    x2 = jnp.clip(pred_ctr_x + 0.5 * pred_w, 0.0, wlim)
    y2 = jnp.clip(pred_ctr_y + 0.5 * pred_h, 0.0, hlim)

    idxr = jax.lax.broadcasted_iota(jnp.int32, (1, np_), 1).astype(f32)
    validr = idxr < float(n)

    neg = jnp.full((1, np_), _NEG, f32)
    mc = jnp.maximum(
        jnp.maximum(jnp.max(jnp.where(validr, x1, neg)),
                    jnp.max(jnp.where(validr, y1, neg))),
        jnp.maximum(jnp.max(jnp.where(validr, x2, neg)),
                    jnp.max(jnp.where(validr, y2, neg)))) + 1.0

    off = id_ref[0:1, :] * mc
    sx1 = x1 + off
    sy1 = y1 + off
    sx2 = x2 + off
    sy2 = y2 + off
    areas = (sx2 - sx1 + 1.0) * (sy2 - sy1 + 1.0)
    sc = sc_ref[0:1, :]

    zero5 = jnp.zeros((5, np_), f32)
    s_ref[...] = jnp.concatenate(
        [sx1, sy1, sx2, sy2, areas, sc, idxr, x1, y1, x2, y2, zero5], axis=0)
    k_ref[...] = jnp.ones((1, np_), f32)

    def dom_tile(ib, keepj_row):
        cols = jnp.transpose(s_ref[:, pl.ds(pl.multiple_of(ib * _BS, _BS), _BS)])
        xi = cols[:, 0:1]
        yi = cols[:, 1:2]
        x2i = cols[:, 2:3]
        y2i = cols[:, 3:4]
        ai = cols[:, 4:5]
        si = cols[:, 5:6]
        ii = cols[:, 6:7]
        xx1 = jnp.maximum(xi, sx1)
        yy1 = jnp.maximum(yi, sy1)
        xx2 = jnp.minimum(x2i, sx2)
        yy2 = jnp.minimum(y2i, sy2)
        inter = (jnp.maximum(xx2 - xx1 + 1.0, 0.0) *
                 jnp.maximum(yy2 - yy1 + 1.0, 0.0))
        iou = inter / (ai + areas - inter)
        hi = (sc > si) | ((sc == si) & (idxr < ii))
        return (iou > thresh) & hi & keepj_row, cols

    def sweep_cond(st):
        return st[1] > 0.0

    def sweep_body(st):
        it = st[0]
        kj = k_ref[0:1, :]

        def blk(ib, _):
            d, _cols = dom_tile(ib, k_ref[0:1, :] > 0.0)
            supp = jnp.max(jnp.where(d, 1.0, 0.0), axis=1, keepdims=True)
            k_ref[0:1, pl.ds(pl.multiple_of(ib * _BS, _BS), _BS)] = (
                jnp.transpose(1.0 - supp))
            return 0

        jax.lax.fori_loop(0, nb, blk, 0)
        knew = k_ref[0:1, :]
        changed = jnp.max(jnp.abs(knew - kj))
        return (it + 1, changed)

    it_f, _ = jax.lax.while_loop(sweep_cond, sweep_body, (jnp.int32(0), f32(1.0)))
    keep = k_ref[0:1, :] > 0.0

    trow = jax.lax.broadcasted_iota(jnp.int32, (1, top_n), 1).astype(f32)

    def blk2(ib, acc):
        base = pl.multiple_of(ib * _BS, _BS)
        cols = jnp.transpose(s_ref[:, pl.ds(base, _BS)])
        si = cols[:, 5:6]
        ii = cols[:, 6:7]
        hi = (sc > si) | ((sc == si) & (idxr < ii))
        pos = jnp.sum(jnp.where(hi & keep, 1.0, 0.0), axis=1, keepdims=True)
        onehot = (pos == trow).astype(f32)
        krow = k_ref[0:1, pl.ds(base, _BS)]
        lane = jax.lax.broadcasted_iota(jnp.int32, (1, _BS), 1) + ib * _BS
        vrow = (lane < n) & (krow > 0.0)
        bj = s_ref[7:11, pl.ds(base, _BS)] * jnp.where(vrow, 1.0, 0.0)
        return acc + jax.lax.dot_general(
            bj, onehot, (((1,), (0,)), ((), ())),
            precision=jax.lax.Precision.HIGHEST, preferred_element_type=f32)

    out_t = jax.lax.fori_loop(0, nb, blk2, jnp.zeros((4, top_n), f32))
    brow = jnp.full((1, top_n), pl.program_id(0).astype(f32))
    out_ref[...] = jnp.transpose(jnp.concatenate([brow, out_t], axis=0))


def kernel(scores, bbox_deltas, im_info, anchors, ids):
    f32 = jnp.float32
    b, n = scores.shape
    np_ = ((n + _BS - 1) // _BS) * _BS
    pad = np_ - n

    sc = jnp.pad(scores.astype(f32), ((0, 0), (0, pad)),
                 constant_values=_NEG)[:, None, :]
    dl = jnp.pad(bbox_deltas.astype(f32),
                 ((0, 0), (0, pad), (0, 0))).transpose(0, 2, 1)
    an = jnp.pad(anchors.astype(f32), ((0, pad), (0, 0))).T
    idr = jnp.pad(ids.astype(f32), (0, pad), constant_values=-1.0)[None, :]
    lim = jnp.stack([im_info[:, 0] - 1.0, im_info[:, 1] - 1.0], axis=1)
    lim = jnp.broadcast_to(lim.astype(f32)[:, :, None], (b, 2, np_))

    kern = functools.partial(_nms_kernel, n=n, np_=np_,
                             top_n=_POST_NMS_TOP_N, thresh=_NMS_THRESH)
    return pl.pallas_call(
        kern,
        grid=(b,),
        in_specs=[
            pl.BlockSpec((None, 1, np_), lambda i: (i, 0, 0)),
            pl.BlockSpec((None, 4, np_), lambda i: (i, 0, 0)),
            pl.BlockSpec((4, np_), lambda i: (0, 0)),
            pl.BlockSpec((1, np_), lambda i: (0, 0)),
            pl.BlockSpec((None, 2, np_), lambda i: (i, 0, 0)),
        ],
        out_specs=pl.BlockSpec((None, _POST_NMS_TOP_N, 5), lambda i: (i, 0, 0)),
        out_shape=jax.ShapeDtypeStruct((b, _POST_NMS_TOP_N, 5), f32),
        scratch_shapes=[
            pltpu.VMEM((16, np_), f32),
            pltpu.VMEM((1, np_), f32),
        ],
        compiler_params=pltpu.CompilerParams(
            dimension_semantics=("parallel",)),
    )(sc, dl, an, idr, lim)


# speedup vs baseline: 65.8617x; 1.1006x over previous
import functools

import jax
import jax.numpy as jnp
from jax.experimental import pallas as pl
from jax.experimental.pallas import tpu as pltpu

_POST_NMS_TOP_N = 1000
_NMS_THRESH = 0.7
_BS = 128
_NEG = -3e38


def _nms_kernel(sc_ref, dl_ref, an_ref, id_ref, lim_ref, out_ref, s_ref, k_ref,
                *, n, np_, top_n, thresh):
    f32 = jnp.float32
    nb = np_ // _BS

    ax1 = an_ref[0:1, :]
    ay1 = an_ref[1:2, :]
    ax2 = an_ref[2:3, :]
    ay2 = an_ref[3:4, :]
    widths = ax2 - ax1 + 1.0
    heights = ay2 - ay1 + 1.0
    ctr_x = ax1 + 0.5 * widths
    ctr_y = ay1 + 0.5 * heights
    dx = dl_ref[0:1, :]
    dy = dl_ref[1:2, :]
    dw = dl_ref[2:3, :]
    dh = dl_ref[3:4, :]
    pred_ctr_x = dx * widths + ctr_x
    pred_ctr_y = dy * heights + ctr_y
    pred_w = jnp.exp(dw) * widths
    pred_h = jnp.exp(dh) * heights
    hlim = lim_ref[0:1, :]
    wlim = lim_ref[1:2, :]
    x1 = jnp.clip(pred_ctr_x - 0.5 * pred_w, 0.0, wlim)
    y1 = jnp.clip(pred_ctr_y - 0.5 * pred_h, 0.0, hlim)
    x2 = jnp.clip(pred_ctr_x + 0.5 * pred_w, 0.0, wlim)
    y2 = jnp.clip(pred_ctr_y + 0.5 * pred_h, 0.0, hlim)

    idxr = jax.lax.broadcasted_iota(jnp.int32, (1, np_), 1).astype(f32)
    validr = idxr < float(n)

    neg = jnp.full((1, np_), _NEG, f32)
    mc = jnp.maximum(
        jnp.maximum(jnp.max(jnp.where(validr, x1, neg)),
                    jnp.max(jnp.where(validr, y1, neg))),
        jnp.maximum(jnp.max(jnp.where(validr, x2, neg)),
                    jnp.max(jnp.where(validr, y2, neg)))) + 1.0

    off = id_ref[0:1, :] * mc
    sx1 = x1 + off
    sy1 = y1 + off
    sx2 = x2 + off
    sy2 = y2 + off
    areas = (sx2 - sx1 + 1.0) * (sy2 - sy1 + 1.0)
    sc = sc_ref[0:1, :]

    zero5 = jnp.zeros((5, np_), f32)
    s_ref[...] = jnp.concatenate(
        [sx1, sy1, sx2, sy2, areas, sc, idxr, x1, y1, x2, y2, zero5], axis=0)
    k_ref[...] = jnp.ones((1, np_), f32)

    trow = jax.lax.broadcasted_iota(jnp.int32, (1, top_n), 1).astype(f32)

    def sweep_cond(st):
        return st[1] > 0.0

    def sweep_body(st):
        it = st[0]
        kj = k_ref[0:1, :]

        def blk(ib, acc):
            base = pl.multiple_of(ib * _BS, _BS)
            cols = jnp.transpose(s_ref[:, pl.ds(base, _BS)])
            xi = cols[:, 0:1]
            yi = cols[:, 1:2]
            x2i = cols[:, 2:3]
            y2i = cols[:, 3:4]
            ai = cols[:, 4:5]
            si = cols[:, 5:6]
            ii = cols[:, 6:7]
            ls = jnp.where(k_ref[0:1, :] > 0.0, sc, _NEG)
            gate = (ls > si) | ((ls == si) & (idxr < ii))
            xx1 = jnp.maximum(xi, sx1)
            yy1 = jnp.maximum(yi, sy1)
            xx2 = jnp.minimum(x2i, sx2)
            yy2 = jnp.minimum(y2i, sy2)
            inter = (jnp.maximum(xx2 - xx1 + 1.0, 0.0) *
                     jnp.maximum(yy2 - yy1 + 1.0, 0.0))
            iou = inter / (ai + areas - inter)
            supp = jnp.any((iou > thresh) & gate, axis=1, keepdims=True)
            knew_row = jnp.transpose(jnp.where(supp, 0.0, 1.0))
            k_ref[0:1, pl.ds(base, _BS)] = knew_row
            pos = jnp.sum(jnp.where(gate, 1.0, 0.0), axis=1, keepdims=True)
            onehot = (pos == trow).astype(f32)
            lane = jax.lax.broadcasted_iota(jnp.int32, (1, _BS), 1) + ib * _BS
            bj = (s_ref[7:11, pl.ds(base, _BS)] * knew_row *
                  jnp.where(lane < n, 1.0, 0.0))
            return acc + jax.lax.dot_general(
                bj, onehot, (((1,), (0,)), ((), ())),
                precision=jax.lax.Precision.HIGHEST, preferred_element_type=f32)

        acc = jax.lax.fori_loop(0, nb, blk, jnp.zeros((4, top_n), f32))
        knew = k_ref[0:1, :]
        changed = jnp.max(jnp.abs(knew - kj))
        return (it + 1, changed, acc)

    _, _, out_t = jax.lax.while_loop(
        sweep_cond, sweep_body,
        (jnp.int32(0), f32(1.0), jnp.zeros((4, top_n), f32)))
    brow = jnp.full((1, top_n), pl.program_id(0).astype(f32))
    out_ref[...] = jnp.transpose(jnp.concatenate([brow, out_t], axis=0))


def kernel(scores, bbox_deltas, im_info, anchors, ids):
    f32 = jnp.float32
    b, n = scores.shape
    np_ = ((n + _BS - 1) // _BS) * _BS
    pad = np_ - n

    sc = jnp.pad(scores.astype(f32), ((0, 0), (0, pad)),
                 constant_values=_NEG)[:, None, :]
    dl = jnp.pad(bbox_deltas.astype(f32),
                 ((0, 0), (0, pad), (0, 0))).transpose(0, 2, 1)
    an = jnp.pad(anchors.astype(f32), ((0, pad), (0, 0))).T
    idr = jnp.pad(ids.astype(f32), (0, pad), constant_values=-1.0)[None, :]
    lim = jnp.stack([im_info[:, 0] - 1.0, im_info[:, 1] - 1.0], axis=1)
    lim = jnp.broadcast_to(lim.astype(f32)[:, :, None], (b, 2, np_))

    kern = functools.partial(_nms_kernel, n=n, np_=np_,
                             top_n=_POST_NMS_TOP_N, thresh=_NMS_THRESH)
    return pl.pallas_call(
        kern,
        grid=(b,),
        in_specs=[
            pl.BlockSpec((None, 1, np_), lambda i: (i, 0, 0)),
            pl.BlockSpec((None, 4, np_), lambda i: (i, 0, 0)),
            pl.BlockSpec((4, np_), lambda i: (0, 0)),
            pl.BlockSpec((1, np_), lambda i: (0, 0)),
            pl.BlockSpec((None, 2, np_), lambda i: (i, 0, 0)),
        ],
        out_specs=pl.BlockSpec((None, _POST_NMS_TOP_N, 5), lambda i: (i, 0, 0)),
        out_shape=jax.ShapeDtypeStruct((b, _POST_NMS_TOP_N, 5), f32),
        scratch_shapes=[
            pltpu.VMEM((16, np_), f32),
            pltpu.VMEM((1, np_), f32),
        ],
        compiler_params=pltpu.CompilerParams(
            dimension_semantics=("parallel",)),
    )(sc, dl, an, idr, lim)


# speedup vs baseline: 66.5760x; 1.0108x over previous
import functools

import jax
import jax.numpy as jnp
from jax.experimental import pallas as pl
from jax.experimental.pallas import tpu as pltpu

_POST_NMS_TOP_N = 1000
_NMS_THRESH = 0.7
_BS = 128
_NEG = -3e38


def _nms_kernel(sc_ref, dl_ref, an_ref, id_ref, lim_ref, out_ref, s_ref, k_ref,
                *, n, np_, top_n, thresh):
    f32 = jnp.float32
    nb = np_ // _BS

    ax1 = an_ref[0:1, :]
    ay1 = an_ref[1:2, :]
    ax2 = an_ref[2:3, :]
    ay2 = an_ref[3:4, :]
    widths = ax2 - ax1 + 1.0
    heights = ay2 - ay1 + 1.0
    ctr_x = ax1 + 0.5 * widths
    ctr_y = ay1 + 0.5 * heights
    dx = dl_ref[0:1, :]
    dy = dl_ref[1:2, :]
    dw = dl_ref[2:3, :]
    dh = dl_ref[3:4, :]
    pred_ctr_x = dx * widths + ctr_x
    pred_ctr_y = dy * heights + ctr_y
    pred_w = jnp.exp(dw) * widths
    pred_h = jnp.exp(dh) * heights
    hlim = lim_ref[0:1, :]
    wlim = lim_ref[1:2, :]
    x1 = jnp.clip(pred_ctr_x - 0.5 * pred_w, 0.0, wlim)
    y1 = jnp.clip(pred_ctr_y - 0.5 * pred_h, 0.0, hlim)
    x2 = jnp.clip(pred_ctr_x + 0.5 * pred_w, 0.0, wlim)
    y2 = jnp.clip(pred_ctr_y + 0.5 * pred_h, 0.0, hlim)

    idxr = jax.lax.broadcasted_iota(jnp.int32, (1, np_), 1).astype(f32)
    validr = idxr < float(n)

    neg = jnp.full((1, np_), _NEG, f32)
    mc = jnp.maximum(
        jnp.maximum(jnp.max(jnp.where(validr, x1, neg)),
                    jnp.max(jnp.where(validr, y1, neg))),
        jnp.maximum(jnp.max(jnp.where(validr, x2, neg)),
                    jnp.max(jnp.where(validr, y2, neg)))) + 1.0

    off = id_ref[0:1, :] * mc
    sx1 = x1 + off
    sy1 = y1 + off
    sx2 = x2 + off
    sy2 = y2 + off
    areas = (sx2 - sx1 + 1.0) * (sy2 - sy1 + 1.0)
    sc = sc_ref[0:1, :]

    zero5 = jnp.zeros((5, np_), f32)
    s_ref[...] = jnp.concatenate(
        [sx1, sy1, sx2, sy2, areas, sc, idxr, x1, y1, x2, y2, zero5], axis=0)
    k_ref[...] = jnp.ones((1, np_), f32)

    def rank_blk(ib, _):
        base = pl.multiple_of(ib * _BS, _BS)
        cols = jnp.transpose(s_ref[:, pl.ds(base, _BS)])
        si = cols[:, 5:6]
        ii = cols[:, 6:7]
        hi = (sc > si) | ((sc == si) & (idxr < ii))
        rank_col = jnp.sum(jnp.where(hi, 1.0, 0.0), axis=1, keepdims=True)
        s_ref[11:12, pl.ds(base, _BS)] = jnp.transpose(rank_col)
        return 0

    jax.lax.fori_loop(0, nb, rank_blk, 0)
    rankr = s_ref[11:12, :]

    trow = jax.lax.broadcasted_iota(jnp.int32, (1, top_n), 1).astype(f32)

    def sweep_cond(st):
        return st[1] > 0.0

    def sweep_body(st):
        it = st[0]
        kj = k_ref[0:1, :]

        def blk(ib, acc):
            base = pl.multiple_of(ib * _BS, _BS)
            cols = jnp.transpose(s_ref[:, pl.ds(base, _BS)])
            xi = cols[:, 0:1]
            yi = cols[:, 1:2]
            x2i = cols[:, 2:3]
            y2i = cols[:, 3:4]
            ai = cols[:, 4:5]
            rk_i = cols[:, 11:12]
            lr = jnp.where(k_ref[0:1, :] > 0.0, rankr, 3e38)
            gate = lr < rk_i
            xx1 = jnp.maximum(xi, sx1)
            yy1 = jnp.maximum(yi, sy1)
            xx2 = jnp.minimum(x2i, sx2)
            yy2 = jnp.minimum(y2i, sy2)
            inter = (jnp.maximum(xx2 - xx1 + 1.0, 0.0) *
                     jnp.maximum(yy2 - yy1 + 1.0, 0.0))
            iou = inter / (ai + areas - inter)
            supp = jnp.any((iou > thresh) & gate, axis=1, keepdims=True)
            knew_row = jnp.transpose(jnp.where(supp, 0.0, 1.0))
            k_ref[0:1, pl.ds(base, _BS)] = knew_row
            pos = jnp.sum(jnp.where(gate, 1.0, 0.0), axis=1, keepdims=True)
            onehot = (pos == trow).astype(f32)
            lane = jax.lax.broadcasted_iota(jnp.int32, (1, _BS), 1) + ib * _BS
            bj = (s_ref[7:11, pl.ds(base, _BS)] * knew_row *
                  jnp.where(lane < n, 1.0, 0.0))
            return acc + jax.lax.dot_general(
                bj, onehot, (((1,), (0,)), ((), ())),
                precision=jax.lax.Precision.HIGHEST, preferred_element_type=f32)

        acc = jax.lax.fori_loop(0, nb, blk, jnp.zeros((4, top_n), f32))
        knew = k_ref[0:1, :]
        changed = jnp.max(jnp.abs(knew - kj))
        return (it + 1, changed, acc)

    _, _, out_t = jax.lax.while_loop(
        sweep_cond, sweep_body,
        (jnp.int32(0), f32(1.0), jnp.zeros((4, top_n), f32)))
    brow = jnp.full((1, top_n), pl.program_id(0).astype(f32))
    out_ref[...] = jnp.transpose(jnp.concatenate([brow, out_t], axis=0))


def kernel(scores, bbox_deltas, im_info, anchors, ids):
    f32 = jnp.float32
    b, n = scores.shape
    np_ = ((n + _BS - 1) // _BS) * _BS
    pad = np_ - n

    sc = jnp.pad(scores.astype(f32), ((0, 0), (0, pad)),
                 constant_values=_NEG)[:, None, :]
    dl = jnp.pad(bbox_deltas.astype(f32),
                 ((0, 0), (0, pad), (0, 0))).transpose(0, 2, 1)
    an = jnp.pad(anchors.astype(f32), ((0, pad), (0, 0))).T
    idr = jnp.pad(ids.astype(f32), (0, pad), constant_values=-1.0)[None, :]
    lim = jnp.stack([im_info[:, 0] - 1.0, im_info[:, 1] - 1.0], axis=1)
    lim = jnp.broadcast_to(lim.astype(f32)[:, :, None], (b, 2, np_))

    kern = functools.partial(_nms_kernel, n=n, np_=np_,
                             top_n=_POST_NMS_TOP_N, thresh=_NMS_THRESH)
    return pl.pallas_call(
        kern,
        grid=(b,),
        in_specs=[
            pl.BlockSpec((None, 1, np_), lambda i: (i, 0, 0)),
            pl.BlockSpec((None, 4, np_), lambda i: (i, 0, 0)),
            pl.BlockSpec((4, np_), lambda i: (0, 0)),
            pl.BlockSpec((1, np_), lambda i: (0, 0)),
            pl.BlockSpec((None, 2, np_), lambda i: (i, 0, 0)),
        ],
        out_specs=pl.BlockSpec((None, _POST_NMS_TOP_N, 5), lambda i: (i, 0, 0)),
        out_shape=jax.ShapeDtypeStruct((b, _POST_NMS_TOP_N, 5), f32),
        scratch_shapes=[
            pltpu.VMEM((16, np_), f32),
            pltpu.VMEM((1, np_), f32),
        ],
        compiler_params=pltpu.CompilerParams(
            dimension_semantics=("parallel",)),
    )(sc, dl, an, idr, lim)


# speedup vs baseline: 72.2947x; 1.0859x over previous
import functools

import jax
import jax.numpy as jnp
from jax.experimental import pallas as pl
from jax.experimental.pallas import tpu as pltpu

_POST_NMS_TOP_N = 1000
_NMS_THRESH = 0.7
_BS = 128
_NEG = -3e38


def _nms_kernel(sc_ref, dl_ref, an_ref, id_ref, lim_ref, out_ref, s_ref, k_ref,
                *, n, np_, top_n, thresh):
    f32 = jnp.float32
    nb = np_ // _BS

    ax1 = an_ref[0:1, :]
    ay1 = an_ref[1:2, :]
    ax2 = an_ref[2:3, :]
    ay2 = an_ref[3:4, :]
    widths = ax2 - ax1 + 1.0
    heights = ay2 - ay1 + 1.0
    ctr_x = ax1 + 0.5 * widths
    ctr_y = ay1 + 0.5 * heights
    dx = dl_ref[0:1, :]
    dy = dl_ref[1:2, :]
    dw = dl_ref[2:3, :]
    dh = dl_ref[3:4, :]
    pred_ctr_x = dx * widths + ctr_x
    pred_ctr_y = dy * heights + ctr_y
    pred_w = jnp.exp(dw) * widths
    pred_h = jnp.exp(dh) * heights
    hlim = lim_ref[0:1, :]
    wlim = lim_ref[1:2, :]
    x1 = jnp.clip(pred_ctr_x - 0.5 * pred_w, 0.0, wlim)
    y1 = jnp.clip(pred_ctr_y - 0.5 * pred_h, 0.0, hlim)
    x2 = jnp.clip(pred_ctr_x + 0.5 * pred_w, 0.0, wlim)
    y2 = jnp.clip(pred_ctr_y + 0.5 * pred_h, 0.0, hlim)

    idxr = jax.lax.broadcasted_iota(jnp.int32, (1, np_), 1).astype(f32)
    validr = idxr < float(n)

    neg = jnp.full((1, np_), _NEG, f32)
    mc = jnp.maximum(
        jnp.maximum(jnp.max(jnp.where(validr, x1, neg)),
                    jnp.max(jnp.where(validr, y1, neg))),
        jnp.maximum(jnp.max(jnp.where(validr, x2, neg)),
                    jnp.max(jnp.where(validr, y2, neg)))) + 1.0

    off = id_ref[0:1, :] * mc
    sx1 = x1 + off
    sy1 = y1 + off
    sx2 = x2 + off
    sy2 = y2 + off
    areas = (sx2 - sx1 + 1.0) * (sy2 - sy1 + 1.0)
    sc = sc_ref[0:1, :]

    zero5 = jnp.zeros((5, np_), f32)
    s_ref[...] = jnp.concatenate(
        [sx1, sy1, sx2, sy2, areas, sc, idxr, x1, y1, x2, y2, zero5], axis=0)
    k_ref[...] = jnp.ones((1, np_), f32)

    def rank_blk(ib, _):
        base = pl.multiple_of(ib * _BS, _BS)
        cols = jnp.transpose(s_ref[:, pl.ds(base, _BS)])
        si = cols[:, 5:6]
        ii = cols[:, 6:7]
        hi = (sc > si) | ((sc == si) & (idxr < ii))
        rank_col = jnp.sum(jnp.where(hi, 1.0, 0.0), axis=1, keepdims=True)
        s_ref[11:12, pl.ds(base, _BS)] = jnp.transpose(rank_col)
        return 0

    jax.lax.fori_loop(0, nb, rank_blk, 0)
    rankr = s_ref[11:12, :]

    trow = jax.lax.broadcasted_iota(jnp.int32, (1, top_n), 1).astype(f32)

    def sweep_cond(st):
        return st[1] > 0.0

    def sweep_body(st):
        it = st[0]
        kj = k_ref[0:1, :]

        def blk(ib, acc):
            base = pl.multiple_of(ib * _BS, _BS)
            cols = jnp.transpose(s_ref[:, pl.ds(base, _BS)])
            xi = cols[:, 0:1]
            yi = cols[:, 1:2]
            x2i = cols[:, 2:3]
            y2i = cols[:, 3:4]
            ai = cols[:, 4:5]
            rk_i = cols[:, 11:12]
            lr = jnp.where(k_ref[0:1, :] > 0.0, rankr, 3e38)
            gate = lr < rk_i
            xx1 = jnp.maximum(xi, sx1)
            yy1 = jnp.maximum(yi, sy1)
            xx2 = jnp.minimum(x2i, sx2)
            yy2 = jnp.minimum(y2i, sy2)
            inter = (jnp.maximum(xx2 - xx1 + 1.0, 0.0) *
                     jnp.maximum(yy2 - yy1 + 1.0, 0.0))
            iou = inter / (ai + areas - inter)
            supp = jnp.any((iou > thresh) & gate, axis=1, keepdims=True)
            knew_row = jnp.transpose(jnp.where(supp, 0.0, 1.0))
            k_ref[0:1, pl.ds(base, _BS)] = knew_row
            pos = jnp.sum(jnp.where(gate, 1.0, 0.0), axis=1, keepdims=True)
            onehot = (pos == trow).astype(f32)
            lane = jax.lax.broadcasted_iota(jnp.int32, (1, _BS), 1) + ib * _BS
            bj = (s_ref[7:11, pl.ds(base, _BS)] * knew_row *
                  jnp.where(lane < n, 1.0, 0.0))
            return acc + jax.lax.dot_general(
                bj, onehot, (((1,), (0,)), ((), ())),
                precision=jax.lax.Precision.HIGHEST, preferred_element_type=f32)

        acc = jax.lax.fori_loop(0, nb, blk, jnp.zeros((4, top_n), f32),
                                unroll=2)
        knew = k_ref[0:1, :]
        changed = jnp.max(jnp.abs(knew - kj))
        return (it + 1, changed, acc)

    _, _, out_t = jax.lax.while_loop(
        sweep_cond, sweep_body,
        (jnp.int32(0), f32(1.0), jnp.zeros((4, top_n), f32)))
    brow = jnp.full((1, top_n), pl.program_id(0).astype(f32))
    out_ref[...] = jnp.transpose(jnp.concatenate([brow, out_t], axis=0))


def kernel(scores, bbox_deltas, im_info, anchors, ids):
    f32 = jnp.float32
    b, n = scores.shape
    np_ = ((n + _BS - 1) // _BS) * _BS
    pad = np_ - n

    sc = jnp.pad(scores.astype(f32), ((0, 0), (0, pad)),
                 constant_values=_NEG)[:, None, :]
    dl = jnp.pad(bbox_deltas.astype(f32),
                 ((0, 0), (0, pad), (0, 0))).transpose(0, 2, 1)
    an = jnp.pad(anchors.astype(f32), ((0, pad), (0, 0))).T
    idr = jnp.pad(ids.astype(f32), (0, pad), constant_values=-1.0)[None, :]
    lim = jnp.stack([im_info[:, 0] - 1.0, im_info[:, 1] - 1.0], axis=1)
    lim = jnp.broadcast_to(lim.astype(f32)[:, :, None], (b, 2, np_))

    kern = functools.partial(_nms_kernel, n=n, np_=np_,
                             top_n=_POST_NMS_TOP_N, thresh=_NMS_THRESH)
    return pl.pallas_call(
        kern,
        grid=(b,),
        in_specs=[
            pl.BlockSpec((None, 1, np_), lambda i: (i, 0, 0)),
            pl.BlockSpec((None, 4, np_), lambda i: (i, 0, 0)),
            pl.BlockSpec((4, np_), lambda i: (0, 0)),
            pl.BlockSpec((1, np_), lambda i: (0, 0)),
            pl.BlockSpec((None, 2, np_), lambda i: (i, 0, 0)),
        ],
        out_specs=pl.BlockSpec((None, _POST_NMS_TOP_N, 5), lambda i: (i, 0, 0)),
        out_shape=jax.ShapeDtypeStruct((b, _POST_NMS_TOP_N, 5), f32),
        scratch_shapes=[
            pltpu.VMEM((16, np_), f32),
            pltpu.VMEM((1, np_), f32),
        ],
        compiler_params=pltpu.CompilerParams(
            dimension_semantics=("parallel",)),
    )(sc, dl, an, idr, lim)


# speedup vs baseline: 73.0077x; 1.0099x over previous
import functools

import jax
import jax.numpy as jnp
from jax.experimental import pallas as pl
from jax.experimental.pallas import tpu as pltpu

_POST_NMS_TOP_N = 1000
_NMS_THRESH = 0.7
_BS = 128
_NEG = -3e38


def _nms_kernel(sc_ref, dl_ref, an_ref, id_ref, lim_ref, out_ref, s_ref, k_ref,
                *, n, np_, top_n, thresh):
    f32 = jnp.float32
    nb = np_ // _BS

    ax1 = an_ref[0:1, :]
    ay1 = an_ref[1:2, :]
    ax2 = an_ref[2:3, :]
    ay2 = an_ref[3:4, :]
    widths = ax2 - ax1 + 1.0
    heights = ay2 - ay1 + 1.0
    ctr_x = ax1 + 0.5 * widths
    ctr_y = ay1 + 0.5 * heights
    dx = dl_ref[0:1, :]
    dy = dl_ref[1:2, :]
    dw = dl_ref[2:3, :]
    dh = dl_ref[3:4, :]
    pred_ctr_x = dx * widths + ctr_x
    pred_ctr_y = dy * heights + ctr_y
    pred_w = jnp.exp(dw) * widths
    pred_h = jnp.exp(dh) * heights
    hlim = lim_ref[0:1, :]
    wlim = lim_ref[1:2, :]
    x1 = jnp.clip(pred_ctr_x - 0.5 * pred_w, 0.0, wlim)
    y1 = jnp.clip(pred_ctr_y - 0.5 * pred_h, 0.0, hlim)
    x2 = jnp.clip(pred_ctr_x + 0.5 * pred_w, 0.0, wlim)
    y2 = jnp.clip(pred_ctr_y + 0.5 * pred_h, 0.0, hlim)

    idxr = jax.lax.broadcasted_iota(jnp.int32, (1, np_), 1).astype(f32)
    validr = idxr < float(n)

    neg = jnp.full((1, np_), _NEG, f32)
    mc = jnp.maximum(
        jnp.maximum(jnp.max(jnp.where(validr, x1, neg)),
                    jnp.max(jnp.where(validr, y1, neg))),
        jnp.maximum(jnp.max(jnp.where(validr, x2, neg)),
                    jnp.max(jnp.where(validr, y2, neg)))) + 1.0

    off = id_ref[0:1, :] * mc
    sx1 = x1 + off
    sy1 = y1 + off
    sx2 = x2 + off
    sy2 = y2 + off
    areas = (sx2 - sx1 + 1.0) * (sy2 - sy1 + 1.0)
    sc = sc_ref[0:1, :]

    zero5 = jnp.zeros((5, np_), f32)
    s_ref[...] = jnp.concatenate(
        [sx1, sy1, sx2, sy2, areas, sc, idxr, x1, y1, x2, y2, zero5], axis=0)
    k_ref[...] = jnp.ones((1, np_), f32)

    def rank_blk(ib, _):
        base = pl.multiple_of(ib * _BS, _BS)
        cols = jnp.transpose(s_ref[:, pl.ds(base, _BS)])
        si = cols[:, 5:6]
        ii = cols[:, 6:7]
        hi = (sc > si) | ((sc == si) & (idxr < ii))
        rank_col = jnp.sum(jnp.where(hi, 1.0, 0.0), axis=1, keepdims=True)
        s_ref[11:12, pl.ds(base, _BS)] = jnp.transpose(rank_col)
        return 0

    jax.lax.fori_loop(0, nb, rank_blk, 0)
    rankr = s_ref[11:12, :]

    trow = jax.lax.broadcasted_iota(jnp.int32, (1, top_n), 1).astype(f32)

    def sweep_cond(st):
        return st[1] > 0.0

    def sweep_body(st):
        it = st[0]
        kj = k_ref[0:1, :]

        def blk(ib, acc):
            base = pl.multiple_of(ib * _BS, _BS)
            cols = jnp.transpose(s_ref[:, pl.ds(base, _BS)])
            xi = cols[:, 0:1]
            yi = cols[:, 1:2]
            x2i = cols[:, 2:3]
            y2i = cols[:, 3:4]
            ai = cols[:, 4:5]
            rk_i = cols[:, 11:12]
            lr = jnp.where(k_ref[0:1, :] > 0.0, rankr, 3e38)
            gate = lr < rk_i
            xx1 = jnp.maximum(xi, sx1)
            yy1 = jnp.maximum(yi, sy1)
            xx2 = jnp.minimum(x2i, sx2)
            yy2 = jnp.minimum(y2i, sy2)
            inter = (jnp.maximum(xx2 - xx1 + 1.0, 0.0) *
                     jnp.maximum(yy2 - yy1 + 1.0, 0.0))
            iou = inter / (ai + areas - inter)
            supp = jnp.any((iou > thresh) & gate, axis=1, keepdims=True)
            knew_row = jnp.transpose(jnp.where(supp, 0.0, 1.0))
            k_ref[0:1, pl.ds(base, _BS)] = knew_row
            pos = jnp.sum(jnp.where(gate, 1.0, 0.0), axis=1, keepdims=True)
            onehot = (pos == trow).astype(f32)
            lane = jax.lax.broadcasted_iota(jnp.int32, (1, _BS), 1) + ib * _BS
            bj = (s_ref[7:11, pl.ds(base, _BS)] * knew_row *
                  jnp.where(lane < n, 1.0, 0.0))
            return acc + jax.lax.dot_general(
                bj, onehot, (((1,), (0,)), ((), ())),
                precision=jax.lax.Precision.HIGHEST, preferred_element_type=f32)

        acc = jax.lax.fori_loop(0, nb, blk, jnp.zeros((4, top_n), f32),
                                unroll=4)
        knew = k_ref[0:1, :]
        changed = jnp.max(jnp.abs(knew - kj))
        return (it + 1, changed, acc)

    _, _, out_t = jax.lax.while_loop(
        sweep_cond, sweep_body,
        (jnp.int32(0), f32(1.0), jnp.zeros((4, top_n), f32)))
    brow = jnp.full((1, top_n), pl.program_id(0).astype(f32))
    out_ref[...] = jnp.transpose(jnp.concatenate([brow, out_t], axis=0))


def kernel(scores, bbox_deltas, im_info, anchors, ids):
    f32 = jnp.float32
    b, n = scores.shape
    np_ = ((n + _BS - 1) // _BS) * _BS
    pad = np_ - n

    sc = jnp.pad(scores.astype(f32), ((0, 0), (0, pad)),
                 constant_values=_NEG)[:, None, :]
    dl = jnp.pad(bbox_deltas.astype(f32),
                 ((0, 0), (0, pad), (0, 0))).transpose(0, 2, 1)
    an = jnp.pad(anchors.astype(f32), ((0, pad), (0, 0))).T
    idr = jnp.pad(ids.astype(f32), (0, pad), constant_values=-1.0)[None, :]
    lim = jnp.stack([im_info[:, 0] - 1.0, im_info[:, 1] - 1.0], axis=1)
    lim = jnp.broadcast_to(lim.astype(f32)[:, :, None], (b, 2, np_))

    kern = functools.partial(_nms_kernel, n=n, np_=np_,
                             top_n=_POST_NMS_TOP_N, thresh=_NMS_THRESH)
    return pl.pallas_call(
        kern,
        grid=(b,),
        in_specs=[
            pl.BlockSpec((None, 1, np_), lambda i: (i, 0, 0)),
            pl.BlockSpec((None, 4, np_), lambda i: (i, 0, 0)),
            pl.BlockSpec((4, np_), lambda i: (0, 0)),
            pl.BlockSpec((1, np_), lambda i: (0, 0)),
            pl.BlockSpec((None, 2, np_), lambda i: (i, 0, 0)),
        ],
        out_specs=pl.BlockSpec((None, _POST_NMS_TOP_N, 5), lambda i: (i, 0, 0)),
        out_shape=jax.ShapeDtypeStruct((b, _POST_NMS_TOP_N, 5), f32),
        scratch_shapes=[
            pltpu.VMEM((16, np_), f32),
            pltpu.VMEM((1, np_), f32),
        ],
        compiler_params=pltpu.CompilerParams(
            dimension_semantics=("parallel",)),
    )(sc, dl, an, idr, lim)
